# baseline (speedup 1.0000x reference)
import functools

import jax
import jax.numpy as jnp
from jax import lax
from jax.experimental import pallas as pl
from jax.experimental.pallas import tpu as pltpu
from jax.experimental.pallas import tpu_sc as plsc

B = 4096
M = 200
D = 64
R = 2 * M
NCHUNK = 5
CHUNK = 80
NC = 2
NS = 16
NW = NC * NS
C = B // NW
NL = 16
NSYM = 1000000


def _sc_bag_body(table_hbm, idx_hbm, out_hbm, idx_v, rows_v, out_v, sem):
    wid = lax.axis_index("s") * NC + lax.axis_index("c")
    base = wid * C

    pltpu.sync_copy(idx_hbm.at[pl.ds(base, C)], idx_v)

    def fire(i, slot):
        for j in range(NCHUNK):
            pltpu.async_copy(
                table_hbm.at[idx_v.at[i, j]],
                rows_v.at[slot, pl.ds(j * CHUNK, CHUNK)],
                sem,
            )

    def drain(i, slot):
        for j in range(NCHUNK):
            pltpu.make_async_copy(
                table_hbm.at[idx_v.at[i, j]],
                rows_v.at[slot, pl.ds(j * CHUNK, CHUNK)],
                sem,
            ).wait()

    def reduce_item(i, slot):
        zero = jnp.zeros((NL,), jnp.float32)

        def red(u, accs):
            new = list(accs)
            for pp in range(2):
                t = 2 * u + pp
                for half in range(2):
                    r = 2 * t + half
                    for k in range(4):
                        a = pp * 8 + half * 4 + k
                        new[a] = new[a] + rows_v[slot, r, pl.ds(k * NL, NL)]
            return tuple(new)

        accs = lax.fori_loop(0, M // 2, red, (zero,) * 16)
        for p in range(8):
            out_v[i, pl.ds(p * NL, NL)] = accs[p] + accs[8 + p]

    fire(0, 0)

    def body2(ii, _):
        i0 = 2 * ii
        fire(i0 + 1, 1)
        drain(i0, 0)
        reduce_item(i0, 0)

        @pl.when(i0 + 2 < C)
        def _():
            fire(i0 + 2, 0)

        drain(i0 + 1, 1)
        reduce_item(i0 + 1, 1)
        return 0

    lax.fori_loop(0, C // 2, body2, 0)
    pltpu.sync_copy(out_v, out_hbm.at[pl.ds(base, C)])


@jax.jit
def _sc_bag(symbol_emb, idx):
    mesh = plsc.VectorSubcoreMesh(core_axis_name="c", subcore_axis_name="s")
    return pl.kernel(
        _sc_bag_body,
        out_type=jax.ShapeDtypeStruct((B, 2 * D), jnp.float32),
        mesh=mesh,
        scratch_types=[
            pltpu.VMEM((C, NCHUNK, CHUNK), jnp.int32),
            pltpu.VMEM((2, R, D), jnp.float32),
            pltpu.VMEM((C, 2 * D), jnp.float32),
            pltpu.SemaphoreType.DMA,
        ],
        compiler_params=pltpu.CompilerParams(use_tc_tiling_on_sc=False),
    )(symbol_emb, idx)


def _tc_body(acc_ref, w_ref, b_ref, n_ref, o_ref):
    z = jnp.dot(acc_ref[...], w_ref[...], preferred_element_type=jnp.float32)
    o_ref[...] = jnp.tanh((z + b_ref[...]) / n_ref[...])


def kernel(symbol_emb, gcn_w_weight, gcn_w_bias, connections, num_neighbors):
    wide = lax.optimization_barrier(
        jnp.pad(symbol_emb[:NSYM], ((0, 0), (0, D))))
    tab = wide.reshape(2 * NSYM, D)
    idx = (connections * 2).reshape(B, NCHUNK, CHUNK)
    acc = _sc_bag(tab, idx)
    wt = gcn_w_weight.T
    b200 = (gcn_w_bias * float(M)).reshape(1, D)
    n = num_neighbors.astype(jnp.float32).reshape(B, 1)
    return pl.pallas_call(
        _tc_body,
        out_shape=jax.ShapeDtypeStruct((B, D), jnp.float32),
    )(acc, wt, b200, n)


# speedup vs baseline: 1.2276x; 1.2276x over previous
import functools

import jax
import jax.numpy as jnp
from jax import lax
from jax.experimental import pallas as pl
from jax.experimental.pallas import tpu as pltpu
from jax.experimental.pallas import tpu_sc as plsc

B = 4096
M = 200
D = 64
R = 2 * M
NCHUNK = 5
CHUNK = 80
NC = 2
NS = 16
NW = NC * NS
C = B // NW
NL = 16
NSYM = 1000000
V = NSYM + 1


def _sc_bag_body(table_hbm, idx_hbm, out_hbm, idx_v, rows_v, out_v, sem):
    wid = lax.axis_index("s") * NC + lax.axis_index("c")
    base = wid * C

    pltpu.sync_copy(idx_hbm.at[pl.ds(base, C)], idx_v)

    def fire(i, slot):
        for j in range(NCHUNK):
            pltpu.async_copy(
                table_hbm.at[idx_v.at[i, j]],
                rows_v.at[slot, pl.ds(j * CHUNK, CHUNK)],
                sem,
            )

    def drain(i, slot):
        for j in range(NCHUNK):
            pltpu.make_async_copy(
                table_hbm.at[idx_v.at[i, j]],
                rows_v.at[slot, pl.ds(j * CHUNK, CHUNK)],
                sem,
            ).wait()

    def reduce_item(i, slot):
        zero = jnp.zeros((NL,), jnp.float32)

        def red(u, accs):
            new = list(accs)
            for pp in range(2):
                t = 2 * u + pp
                for half in range(2):
                    r = 2 * t + half
                    for k in range(4):
                        a = pp * 8 + half * 4 + k
                        new[a] = new[a] + rows_v[slot, r, pl.ds(k * NL, NL)]
            return tuple(new)

        accs = lax.fori_loop(0, M // 2, red, (zero,) * 16)
        for p in range(8):
            out_v[i, pl.ds(p * NL, NL)] = accs[p] + accs[8 + p]

    fire(0, 0)

    def body2(ii, _):
        i0 = 2 * ii
        fire(i0 + 1, 1)
        drain(i0, 0)
        reduce_item(i0, 0)

        @pl.when(i0 + 2 < C)
        def _():
            fire(i0 + 2, 0)

        drain(i0 + 1, 1)
        reduce_item(i0 + 1, 1)
        return 0

    lax.fori_loop(0, C // 2, body2, 0)
    pltpu.sync_copy(out_v, out_hbm.at[pl.ds(base, C)])


@jax.jit
def _sc_bag(symbol_emb, idx):
    mesh = plsc.VectorSubcoreMesh(core_axis_name="c", subcore_axis_name="s")
    return pl.kernel(
        _sc_bag_body,
        out_type=jax.ShapeDtypeStruct((B, 2 * D), jnp.float32),
        mesh=mesh,
        scratch_types=[
            pltpu.VMEM((C, NCHUNK, CHUNK), jnp.int32),
            pltpu.VMEM((2, R, D), jnp.float32),
            pltpu.VMEM((C, 2 * D), jnp.float32),
            pltpu.SemaphoreType.DMA,
        ],
        compiler_params=pltpu.CompilerParams(use_tc_tiling_on_sc=False),
    )(symbol_emb, idx)


SB = 1024
NBLK = 489
HALF = NBLK * SB


def _tr_body(lo_ref, hi_ref, o_ref):
    o_ref[:, 0:D] = lo_ref[...].T
    o_ref[:, D:2 * D] = hi_ref[...].T


@jax.jit
def _tc_transpose(tab_t):
    return pl.pallas_call(
        _tr_body,
        grid=(NBLK,),
        in_specs=[
            pl.BlockSpec((D, SB), lambda i: (0, i)),
            pl.BlockSpec((D, SB), lambda i: (0, jnp.minimum(i + NBLK,
                                                            V // SB))),
        ],
        out_specs=pl.BlockSpec((SB, 2 * D), lambda i: (i, 0)),
        out_shape=jax.ShapeDtypeStruct((HALF, 2 * D), jnp.float32),
    )(tab_t, tab_t)


def _tc_body(acc_ref, w_ref, b_ref, n_ref, o_ref):
    z = jnp.dot(acc_ref[...], w_ref[...], preferred_element_type=jnp.float32)
    o_ref[...] = jnp.tanh((z + b_ref[...]) / n_ref[...])


def kernel(symbol_emb, gcn_w_weight, gcn_w_bias, connections, num_neighbors):
    pairs = _tc_transpose(symbol_emb.T)
    tab = pairs.reshape(2 * HALF, D)
    idx = jnp.where(connections < HALF, connections * 2,
                    connections * 2 - (2 * HALF - 1))
    idx = idx.reshape(B, NCHUNK, CHUNK)
    acc = _sc_bag(tab, idx)
    wt = gcn_w_weight.T
    b200 = (gcn_w_bias * float(M)).reshape(1, D)
    n = num_neighbors.astype(jnp.float32).reshape(B, 1)
    return pl.pallas_call(
        _tc_body,
        out_shape=jax.ShapeDtypeStruct((B, D), jnp.float32),
    )(acc, wt, b200, n)


# speedup vs baseline: 1.6952x; 1.3809x over previous
import functools

import jax
import jax.numpy as jnp
from jax import lax
from jax.experimental import pallas as pl
from jax.experimental.pallas import tpu as pltpu
from jax.experimental.pallas import tpu_sc as plsc

B = 4096
M = 200
D = 64
R = 2 * M
NCHUNK = 5
CHUNK = 80
NC = 2
NS = 16
NW = NC * NS
C = B // NW
NL = 16
NSYM = 1000000
V = NSYM + 1


def _sc_bag_body(table_hbm, idx_hbm, out_hbm, idx_v, rows_v, out_v, sem):
    wid = lax.axis_index("s") * NC + lax.axis_index("c")
    base = wid * C

    pltpu.sync_copy(idx_hbm.at[pl.ds(base, C)], idx_v)

    def fire(i, slot):
        for j in range(NCHUNK):
            pltpu.async_copy(
                table_hbm.at[idx_v.at[i, j]],
                rows_v.at[slot, pl.ds(j * CHUNK, CHUNK)],
                sem,
            )

    def drain(i, slot):
        for j in range(NCHUNK):
            pltpu.make_async_copy(
                table_hbm.at[idx_v.at[i, j]],
                rows_v.at[slot, pl.ds(j * CHUNK, CHUNK)],
                sem,
            ).wait()

    def reduce_item(i, slot):
        zero = jnp.zeros((NL,), jnp.float32)

        def red(u, accs):
            new = list(accs)
            for pp in range(2):
                t = 2 * u + pp
                for half in range(2):
                    r = 2 * t + half
                    for k in range(4):
                        a = pp * 8 + half * 4 + k
                        new[a] = new[a] + rows_v[slot, r, pl.ds(k * NL, NL)]
            return tuple(new)

        accs = lax.fori_loop(0, M // 2, red, (zero,) * 16)
        for p in range(8):
            out_v[i, pl.ds(p * NL, NL)] = accs[p] + accs[8 + p]

    fire(0, 0)

    def body2(ii, _):
        i0 = 2 * ii
        fire(i0 + 1, 1)
        drain(i0, 0)
        reduce_item(i0, 0)

        @pl.when(i0 + 2 < C)
        def _():
            fire(i0 + 2, 0)

        drain(i0 + 1, 1)
        reduce_item(i0 + 1, 1)
        return 0

    lax.fori_loop(0, C // 2, body2, 0)
    pltpu.sync_copy(out_v, out_hbm.at[pl.ds(base, C)])


@jax.jit
def _sc_bag(symbol_emb, idx):
    mesh = plsc.VectorSubcoreMesh(core_axis_name="c", subcore_axis_name="s")
    return pl.kernel(
        _sc_bag_body,
        out_type=jax.ShapeDtypeStruct((B, 2 * D), jnp.float32),
        mesh=mesh,
        scratch_types=[
            pltpu.VMEM((C, NCHUNK, CHUNK), jnp.int32),
            pltpu.VMEM((2, R, D), jnp.float32),
            pltpu.VMEM((C, 2 * D), jnp.float32),
            pltpu.SemaphoreType.DMA,
        ],
        compiler_params=pltpu.CompilerParams(use_tc_tiling_on_sc=False),
    )(symbol_emb, idx)


SB = 4096
NBLK = 123
HALF = NBLK * SB


def _tr_body(lo_ref, hi_ref, o_ref):
    o_ref[:, 0:D] = lo_ref[...].T
    o_ref[:, D:2 * D] = hi_ref[...].T


@jax.jit
def _tc_transpose(tab_t):
    return pl.pallas_call(
        _tr_body,
        grid=(NBLK,),
        in_specs=[
            pl.BlockSpec((D, SB), lambda i: (0, i)),
            pl.BlockSpec((D, SB), lambda i: (0, jnp.minimum(i + NBLK,
                                                            V // SB))),
        ],
        out_specs=pl.BlockSpec((SB, 2 * D), lambda i: (i, 0)),
        out_shape=jax.ShapeDtypeStruct((HALF, 2 * D), jnp.float32),
    )(tab_t, tab_t)


def _tc_body(acc_ref, w_ref, b_ref, n_ref, o_ref):
    z = jnp.dot(acc_ref[...], w_ref[...], preferred_element_type=jnp.float32)
    o_ref[...] = jnp.tanh((z + b_ref[...]) / n_ref[...])


def kernel(symbol_emb, gcn_w_weight, gcn_w_bias, connections, num_neighbors):
    pairs = _tc_transpose(symbol_emb.T)
    tab = pairs.reshape(2 * HALF, D)
    idx = jnp.where(connections < HALF, connections * 2,
                    connections * 2 - (2 * HALF - 1))
    idx = idx.reshape(B, NCHUNK, CHUNK)
    acc = _sc_bag(tab, idx)
    wt = gcn_w_weight.T
    b200 = (gcn_w_bias * float(M)).reshape(1, D)
    n = num_neighbors.astype(jnp.float32).reshape(B, 1)
    return pl.pallas_call(
        _tc_body,
        out_shape=jax.ShapeDtypeStruct((B, D), jnp.float32),
    )(acc, wt, b200, n)


# speedup vs baseline: 1.8153x; 1.0709x over previous
import functools

import jax
import jax.numpy as jnp
from jax import lax
from jax.experimental import pallas as pl
from jax.experimental.pallas import tpu as pltpu
from jax.experimental.pallas import tpu_sc as plsc

B = 4096
M = 200
D = 64
R = 2 * M
NCHUNK = 5
CHUNK = 80
NC = 2
NS = 16
NW = NC * NS
C = B // NW
NL = 16
NSYM = 1000000
V = NSYM + 1


def _sc_bag_body(table_hbm, idx_hbm, out_hbm, idx_v, rows_v, out_v, sem):
    wid = lax.axis_index("s") * NC + lax.axis_index("c")
    base = wid * C

    pltpu.sync_copy(idx_hbm.at[pl.ds(base, C)], idx_v)

    def fire(i, slot):
        for j in range(NCHUNK):
            pltpu.async_copy(
                table_hbm.at[idx_v.at[i, j]],
                rows_v.at[slot, pl.ds(j * CHUNK, CHUNK)],
                sem,
            )

    def drain(i, slot):
        for j in range(NCHUNK):
            pltpu.make_async_copy(
                table_hbm.at[idx_v.at[i, j]],
                rows_v.at[slot, pl.ds(j * CHUNK, CHUNK)],
                sem,
            ).wait()

    def reduce_item(i, slot):
        zero = jnp.zeros((NL,), jnp.float32)

        def red(u, accs):
            new = list(accs)
            for pp in range(2):
                t = 2 * u + pp
                for half in range(2):
                    r = 2 * t + half
                    for k in range(4):
                        a = pp * 8 + half * 4 + k
                        new[a] = new[a] + rows_v[slot, r, pl.ds(k * NL, NL)]
            return tuple(new)

        accs = lax.fori_loop(0, M // 2, red, (zero,) * 16)
        for p in range(8):
            out_v[i, pl.ds(p * NL, NL)] = accs[p] + accs[8 + p]

    fire(0, 0)

    def body2(ii, _):
        i0 = 2 * ii
        fire(i0 + 1, 1)
        drain(i0, 0)
        reduce_item(i0, 0)

        @pl.when(i0 + 2 < C)
        def _():
            fire(i0 + 2, 0)

        drain(i0 + 1, 1)
        reduce_item(i0 + 1, 1)
        return 0

    lax.fori_loop(0, C // 2, body2, 0)
    pltpu.sync_copy(out_v, out_hbm.at[pl.ds(base, C)])


@jax.jit
def _sc_bag(symbol_emb, idx):
    mesh = plsc.VectorSubcoreMesh(core_axis_name="c", subcore_axis_name="s")
    return pl.kernel(
        _sc_bag_body,
        out_type=jax.ShapeDtypeStruct((B, 2 * D), jnp.float32),
        mesh=mesh,
        scratch_types=[
            pltpu.VMEM((C, NCHUNK, CHUNK), jnp.int32),
            pltpu.VMEM((2, R, D), jnp.float32),
            pltpu.VMEM((C, 2 * D), jnp.float32),
            pltpu.SemaphoreType.DMA,
        ],
        compiler_params=pltpu.CompilerParams(use_tc_tiling_on_sc=False),
    )(symbol_emb, idx)


SB = 8192
NBLK = 62
HALF = NBLK * SB


def _tr_body(lo_ref, hi_ref, o_ref):
    o_ref[:, 0:D] = lo_ref[...].T
    o_ref[:, D:2 * D] = hi_ref[...].T


@jax.jit
def _tc_transpose(tab_t):
    return pl.pallas_call(
        _tr_body,
        grid=(NBLK,),
        in_specs=[
            pl.BlockSpec((D, SB), lambda i: (0, i)),
            pl.BlockSpec((D, SB), lambda i: (0, jnp.minimum(i + NBLK,
                                                            V // SB))),
        ],
        out_specs=pl.BlockSpec((SB, 2 * D), lambda i: (i, 0)),
        out_shape=jax.ShapeDtypeStruct((HALF, 2 * D), jnp.float32),
    )(tab_t, tab_t)


def _tc_body(acc_ref, w_ref, b_ref, n_ref, o_ref):
    z = jnp.dot(acc_ref[...], w_ref[...], preferred_element_type=jnp.float32)
    o_ref[...] = jnp.tanh((z + b_ref[...]) / n_ref[...])


def kernel(symbol_emb, gcn_w_weight, gcn_w_bias, connections, num_neighbors):
    pairs = _tc_transpose(symbol_emb.T)
    tab = pairs.reshape(2 * HALF, D)
    idx = jnp.where(connections < HALF, connections * 2,
                    connections * 2 - (2 * HALF - 1))
    idx = idx.reshape(B, NCHUNK, CHUNK)
    acc = _sc_bag(tab, idx)
    wt = gcn_w_weight.T
    b200 = (gcn_w_bias * float(M)).reshape(1, D)
    n = num_neighbors.astype(jnp.float32).reshape(B, 1)
    return pl.pallas_call(
        _tc_body,
        out_shape=jax.ShapeDtypeStruct((B, D), jnp.float32),
    )(acc, wt, b200, n)


# speedup vs baseline: 1.8827x; 1.0371x over previous
import functools

import jax
import jax.numpy as jnp
from jax import lax
from jax.experimental import pallas as pl
from jax.experimental.pallas import tpu as pltpu
from jax.experimental.pallas import tpu_sc as plsc

B = 4096
M = 200
D = 64
R = 2 * M
NCHUNK = 5
CHUNK = 80
NC = 2
NS = 16
NW = NC * NS
C = B // NW
NL = 16
NSYM = 1000000
V = NSYM + 1


def _sc_bag_body(table_hbm, idx_hbm, out_hbm, idx_v, rows_v, out_v, sem):
    wid = lax.axis_index("s") * NC + lax.axis_index("c")
    base = wid * C

    pltpu.sync_copy(idx_hbm.at[pl.ds(base, C)], idx_v)

    def fire(i, slot):
        for j in range(NCHUNK):
            pltpu.async_copy(
                table_hbm.at[idx_v.at[i, j]],
                rows_v.at[slot, pl.ds(j * CHUNK, CHUNK)],
                sem,
            )

    def drain(i, slot):
        for j in range(NCHUNK):
            pltpu.make_async_copy(
                table_hbm.at[idx_v.at[i, j]],
                rows_v.at[slot, pl.ds(j * CHUNK, CHUNK)],
                sem,
            ).wait()

    def reduce_item(i, slot):
        zero = jnp.zeros((NL,), jnp.float32)

        def red(u, accs):
            new = list(accs)
            for pp in range(2):
                t = 2 * u + pp
                for half in range(2):
                    r = 2 * t + half
                    for k in range(4):
                        a = pp * 8 + half * 4 + k
                        new[a] = new[a] + rows_v[slot, r, pl.ds(k * NL, NL)]
            return tuple(new)

        accs = lax.fori_loop(0, M // 2, red, (zero,) * 16)
        for p in range(8):
            out_v[i, pl.ds(p * NL, NL)] = accs[p] + accs[8 + p]

    fire(0, 0)

    def body2(ii, _):
        i0 = 2 * ii
        fire(i0 + 1, 1)
        drain(i0, 0)
        reduce_item(i0, 0)

        @pl.when(i0 + 2 < C)
        def _():
            fire(i0 + 2, 0)

        drain(i0 + 1, 1)
        reduce_item(i0 + 1, 1)
        return 0

    lax.fori_loop(0, C // 2, body2, 0)
    pltpu.sync_copy(out_v, out_hbm.at[pl.ds(base, C)])


@jax.jit
def _sc_bag(symbol_emb, idx):
    mesh = plsc.VectorSubcoreMesh(core_axis_name="c", subcore_axis_name="s")
    return pl.kernel(
        _sc_bag_body,
        out_type=jax.ShapeDtypeStruct((B, 2 * D), jnp.float32),
        mesh=mesh,
        scratch_types=[
            pltpu.VMEM((C, NCHUNK, CHUNK), jnp.int32),
            pltpu.VMEM((2, R, D), jnp.float32),
            pltpu.VMEM((C, 2 * D), jnp.float32),
            pltpu.SemaphoreType.DMA,
        ],
        compiler_params=pltpu.CompilerParams(use_tc_tiling_on_sc=False),
    )(symbol_emb, idx)


SB = 16384
NBLK = 31
HALF = NBLK * SB


def _tr_body(lo_ref, hi_ref, o_ref):
    o_ref[:, 0:D] = lo_ref[...].T
    o_ref[:, D:2 * D] = hi_ref[...].T


@jax.jit
def _tc_transpose(tab_t):
    return pl.pallas_call(
        _tr_body,
        grid=(NBLK,),
        in_specs=[
            pl.BlockSpec((D, SB), lambda i: (0, i)),
            pl.BlockSpec((D, SB), lambda i: (0, jnp.minimum(i + NBLK,
                                                            V // SB))),
        ],
        out_specs=pl.BlockSpec((SB, 2 * D), lambda i: (i, 0)),
        out_shape=jax.ShapeDtypeStruct((HALF, 2 * D), jnp.float32),
    )(tab_t, tab_t)


def _tc_body(acc_ref, w_ref, b_ref, n_ref, o_ref):
    z = jnp.dot(acc_ref[...], w_ref[...], preferred_element_type=jnp.float32)
    o_ref[...] = jnp.tanh((z + b_ref[...]) / n_ref[...])


def kernel(symbol_emb, gcn_w_weight, gcn_w_bias, connections, num_neighbors):
    pairs = _tc_transpose(symbol_emb.T)
    tab = pairs.reshape(2 * HALF, D)
    idx = jnp.where(connections < HALF, connections * 2,
                    connections * 2 - (2 * HALF - 1))
    idx = idx.reshape(B, NCHUNK, CHUNK)
    acc = _sc_bag(tab, idx)
    wt = gcn_w_weight.T
    b200 = (gcn_w_bias * float(M)).reshape(1, D)
    n = num_neighbors.astype(jnp.float32).reshape(B, 1)
    return pl.pallas_call(
        _tc_body,
        out_shape=jax.ShapeDtypeStruct((B, D), jnp.float32),
    )(acc, wt, b200, n)


# speedup vs baseline: 1.8963x; 1.0072x over previous
import functools

import jax
import jax.numpy as jnp
from jax import lax
from jax.experimental import pallas as pl
from jax.experimental.pallas import tpu as pltpu
from jax.experimental.pallas import tpu_sc as plsc

B = 4096
M = 200
D = 64
R = 2 * M
NCHUNK = 5
CHUNK = 80
NC = 2
NS = 16
NW = NC * NS
C = B // NW
NL = 16
NSYM = 1000000
V = NSYM + 1


def _sc_bag_body(table_hbm, idx_hbm, out_hbm, idx_v, rows_v, out_v, sem):
    wid = lax.axis_index("s") * NC + lax.axis_index("c")
    base = wid * C

    pltpu.sync_copy(idx_hbm.at[pl.ds(base, C)], idx_v)

    def fire(i, slot):
        pltpu.async_copy(table_hbm.at[idx_v.at[i]], rows_v.at[slot], sem)

    def drain(i, slot):
        pltpu.make_async_copy(table_hbm.at[idx_v.at[i]], rows_v.at[slot],
                              sem).wait()

    def reduce_item(i, slot):
        zero = jnp.zeros((NL,), jnp.float32)

        def red(u, accs):
            new = list(accs)
            for pp in range(2):
                t = 2 * u + pp
                for half in range(2):
                    r = 2 * t + half
                    for k in range(4):
                        a = pp * 8 + half * 4 + k
                        new[a] = new[a] + rows_v[slot, r, pl.ds(k * NL, NL)]
            return tuple(new)

        accs = lax.fori_loop(0, M // 2, red, (zero,) * 16)
        for p in range(8):
            out_v[i, pl.ds(p * NL, NL)] = accs[p] + accs[8 + p]

    fire(0, 0)

    def body2(ii, _):
        i0 = 2 * ii
        fire(i0 + 1, 1)
        drain(i0, 0)
        reduce_item(i0, 0)

        @pl.when(i0 + 2 < C)
        def _():
            fire(i0 + 2, 0)

        drain(i0 + 1, 1)
        reduce_item(i0 + 1, 1)
        return 0

    lax.fori_loop(0, C // 2, body2, 0)
    pltpu.sync_copy(out_v, out_hbm.at[pl.ds(base, C)])


@jax.jit
def _sc_bag(symbol_emb, idx):
    mesh = plsc.VectorSubcoreMesh(core_axis_name="c", subcore_axis_name="s")
    return pl.kernel(
        _sc_bag_body,
        out_type=jax.ShapeDtypeStruct((B, 2 * D), jnp.float32),
        mesh=mesh,
        scratch_types=[
            pltpu.VMEM((C, R), jnp.int32),
            pltpu.VMEM((2, R, D), jnp.float32),
            pltpu.VMEM((C, 2 * D), jnp.float32),
            pltpu.SemaphoreType.DMA,
        ],
        compiler_params=pltpu.CompilerParams(use_tc_tiling_on_sc=False),
    )(symbol_emb, idx)


SB = 16384
NBLK = 31
HALF = NBLK * SB


def _tr_body(lo_ref, hi_ref, o_ref):
    o_ref[:, 0:D] = lo_ref[...].T
    o_ref[:, D:2 * D] = hi_ref[...].T


@jax.jit
def _tc_transpose(tab_t):
    return pl.pallas_call(
        _tr_body,
        grid=(NBLK,),
        in_specs=[
            pl.BlockSpec((D, SB), lambda i: (0, i)),
            pl.BlockSpec((D, SB), lambda i: (0, jnp.minimum(i + NBLK,
                                                            V // SB))),
        ],
        out_specs=pl.BlockSpec((SB, 2 * D), lambda i: (i, 0)),
        out_shape=jax.ShapeDtypeStruct((HALF, 2 * D), jnp.float32),
    )(tab_t, tab_t)


def _tc_body(acc_ref, w_ref, b_ref, n_ref, o_ref):
    z = jnp.dot(acc_ref[...], w_ref[...], preferred_element_type=jnp.float32)
    o_ref[...] = jnp.tanh((z + b_ref[...]) / n_ref[...])


def kernel(symbol_emb, gcn_w_weight, gcn_w_bias, connections, num_neighbors):
    pairs = _tc_transpose(symbol_emb.T)
    tab = pairs.reshape(2 * HALF, D)
    idx = jnp.where(connections < HALF, connections * 2,
                    connections * 2 - (2 * HALF - 1))
    idx = idx.reshape(B, R)
    acc = _sc_bag(tab, idx)
    wt = gcn_w_weight.T
    b200 = (gcn_w_bias * float(M)).reshape(1, D)
    n = num_neighbors.astype(jnp.float32).reshape(B, 1)
    return pl.pallas_call(
        _tc_body,
        out_shape=jax.ShapeDtypeStruct((B, D), jnp.float32),
    )(acc, wt, b200, n)


# speedup vs baseline: 1.8987x; 1.0012x over previous
import jax
import jax.numpy as jnp
from jax import lax
from jax.experimental import pallas as pl
from jax.experimental.pallas import tpu as pltpu
from jax.experimental.pallas import tpu_sc as plsc

B = 4096
M = 200
D = 64
R = 2 * M
NC = 2
NS = 16
NW = NC * NS
C = B // NW
NL = 16
NSYM = 1000000
V = NSYM + 1


def _sc_bag_body(table_hbm, idx_hbm, out_hbm, idx_v, rows_v, out_v, sem):
    wid = lax.axis_index("s") * NC + lax.axis_index("c")
    base = wid * C

    pltpu.sync_copy(idx_hbm.at[pl.ds(base, C)], idx_v)

    def fire(i, slot):
        pltpu.async_copy(table_hbm.at[idx_v.at[i]], rows_v.at[slot], sem)

    def drain(i, slot):
        pltpu.make_async_copy(table_hbm.at[idx_v.at[i]], rows_v.at[slot],
                              sem).wait()

    def reduce_item(i, slot):
        zero = jnp.zeros((NL,), jnp.float32)

        def red(u, accs):
            new = list(accs)
            for pp in range(2):
                t = 2 * u + pp
                for half in range(2):
                    r = 2 * t + half
                    for k in range(4):
                        a = pp * 8 + half * 4 + k
                        new[a] = new[a] + rows_v[slot, r, pl.ds(k * NL, NL)]
            return tuple(new)

        accs = lax.fori_loop(0, M // 2, red, (zero,) * 16)
        for p in range(8):
            out_v[i, pl.ds(p * NL, NL)] = accs[p] + accs[8 + p]

    fire(0, 0)

    def body2(ii, _):
        i0 = 2 * ii
        fire(i0 + 1, 1)
        drain(i0, 0)
        reduce_item(i0, 0)

        @pl.when(i0 + 2 < C)
        def _():
            fire(i0 + 2, 0)

        drain(i0 + 1, 1)
        reduce_item(i0 + 1, 1)
        return 0

    lax.fori_loop(0, C // 2, body2, 0)
    pltpu.sync_copy(out_v, out_hbm.at[pl.ds(base, C)])


@jax.jit
def _sc_bag(symbol_emb, idx):
    mesh = plsc.VectorSubcoreMesh(core_axis_name="c", subcore_axis_name="s")
    return pl.kernel(
        _sc_bag_body,
        out_type=jax.ShapeDtypeStruct((B, 2 * D), jnp.float32),
        mesh=mesh,
        scratch_types=[
            pltpu.VMEM((C, R), jnp.int32),
            pltpu.VMEM((2, R, D), jnp.float32),
            pltpu.VMEM((C, 2 * D), jnp.float32),
            pltpu.SemaphoreType.DMA,
        ],
        compiler_params=pltpu.CompilerParams(use_tc_tiling_on_sc=False),
    )(symbol_emb, idx)


SB = 16384
NBLK = 31
HALF = NBLK * SB


def _tr_body(lo_ref, hi_ref, o_ref):
    o_ref[:, 0:D] = lo_ref[...].T
    o_ref[:, D:2 * D] = hi_ref[...].T


@jax.jit
def _tc_transpose(tab_t):
    return pl.pallas_call(
        _tr_body,
        grid=(NBLK,),
        in_specs=[
            pl.BlockSpec((D, SB), lambda i: (0, i)),
            pl.BlockSpec((D, SB), lambda i: (0, jnp.minimum(i + NBLK,
                                                            V // SB))),
        ],
        out_specs=pl.BlockSpec((SB, 2 * D), lambda i: (i, 0)),
        out_shape=jax.ShapeDtypeStruct((HALF, 2 * D), jnp.float32),
    )(tab_t, tab_t)


def _tc_body(acc_ref, w_ref, b_ref, n_ref, o_ref):
    z = jnp.dot(acc_ref[...], w_ref[...], preferred_element_type=jnp.float32)
    o_ref[...] = jnp.tanh((z + b_ref[...]) / n_ref[...])


def kernel(symbol_emb, gcn_w_weight, gcn_w_bias, connections, num_neighbors):
    pairs = _tc_transpose(symbol_emb.T)
    tab = pairs.reshape(2 * HALF, D)
    idx = jnp.where(connections < HALF, connections * 2,
                    connections * 2 - (2 * HALF - 1))
    idx = idx.reshape(B, R)
    acc = _sc_bag(tab, idx)
    wt = gcn_w_weight.T
    b200 = (gcn_w_bias * float(M)).reshape(1, D)
    n = num_neighbors.astype(jnp.float32).reshape(B, 1)
    return pl.pallas_call(
        _tc_body,
        out_shape=jax.ShapeDtypeStruct((B, D), jnp.float32),
    )(acc, wt, b200, n)
